# baseline (speedup 1.0000x reference)
import functools

import jax
import jax.numpy as jnp
from jax import lax
from jax.experimental import pallas as pl
from jax.experimental.pallas import tpu as pltpu
from jax.experimental.pallas import tpu_sc as plsc

N = 50000
E = 800000
B = 256
L = 730
D_IN = 78
H = 128

_PREC = jax.lax.Precision.HIGHEST

QROWS = 8448
NQ = 6
NPAD = NQ * QROWS
UROWS = 50048
ZROW = 50000
TRASH = QROWS
BUFROWS = QROWS + 16
KB = 128
EPAD = 850048
NBLK = EPAD // KB
TROWS = QROWS // 16
ZROWS = 64
ZCHUNKS = [(o, min(ZROWS, TROWS - o)) for o in range(0, TROWS, ZROWS)]


def _propagate_body(u_hbm, srcs_hbm, dsts_hbm, tab_hbm, out_hbm,
                    tab_v, src_v, dst_v, idxl_v, rows_v, zero_v, buf_sh, sem):
    c = lax.axis_index("c")
    s = lax.axis_index("s")
    pltpu.sync_copy(tab_hbm, tab_v)
    tabs = [tab_v[pl.ds(16 * q, 16)] for q in range(NQ)]
    zvec = jnp.zeros((16,), jnp.float32)

    @pl.loop(0, ZROWS)
    def _(r):
        @pl.loop(0, H, step=16)
        def _(f):
            zero_v[r, pl.ds(f, 16)] = zvec

    for qi in range(NQ // 2):
        q = (NQ // 2) * c + qi
        qbase = q * QROWS
        tq = jnp.where(c == 0, tabs[qi], tabs[NQ // 2 + qi])
        sblk = tq[0]
        nblk = tq[1]

        for zo, zn in ZCHUNKS:
            zoff = pl.multiple_of(s * TROWS + zo, 8)
            pltpu.sync_copy(zero_v.at[pl.ds(0, zn)],
                            buf_sh.at[pl.ds(zoff, zn)])

        plsc.subcore_barrier()

        nmine = (nblk - s + 15) // 16

        @pl.loop(0, nmine)
        def _(i):
            blk = sblk + s + i * 16
            off = pl.multiple_of(blk * KB, KB)
            pltpu.sync_copy(srcs_hbm.at[pl.ds(off, KB)], src_v)
            pltpu.sync_copy(dsts_hbm.at[pl.ds(off, KB)], dst_v)
            for j in range(KB // 16):
                d = dst_v[pl.ds(j * 16, 16)]
                in_q = (d >= qbase) & (d < qbase + QROWS)
                loc = jnp.where(in_q, d - qbase, TRASH)
                idxl_v[pl.ds(j * 16, 16)] = loc
            pltpu.async_copy(u_hbm.at[src_v], rows_v, sem).wait()
            pltpu.sync_copy(rows_v, buf_sh.at[idxl_v], add=True)

        plsc.subcore_barrier()

        pltpu.sync_copy(buf_sh.at[pl.ds(pl.multiple_of(s * TROWS, 8), TROWS)],
                        out_hbm.at[pl.ds(pl.multiple_of(qbase + s * TROWS, 8), TROWS)])


def _propagate(u, srcs, dsts, tab):
    mesh = plsc.VectorSubcoreMesh(core_axis_name="c", subcore_axis_name="s")
    kern = pl.kernel(
        _propagate_body,
        out_type=jax.ShapeDtypeStruct((NPAD, H), jnp.float32),
        mesh=mesh,
        scratch_types=[
            pltpu.VMEM((16 * NQ,), jnp.int32),
            pltpu.VMEM((KB,), jnp.int32),
            pltpu.VMEM((KB,), jnp.int32),
            pltpu.VMEM((KB,), jnp.int32),
            pltpu.VMEM((KB, H), jnp.float32),
            pltpu.VMEM((ZROWS, H), jnp.float32),
            pltpu.VMEM_SHARED((BUFROWS, H), jnp.float32),
            pltpu.SemaphoreType.DMA,
        ],
    )
    return kern(u, srcs, dsts, tab)


def _bn_rows(x, g, b):
    m = jnp.mean(x, 0)
    v = jnp.mean((x - m) ** 2, 0)
    return g * (x - m) / jnp.sqrt(v + 1e-5) + b


def _head_body(zp_ref, xt_ref, w1_ref, b1_ref, g1_ref, bb1_ref,
               w2_ref, b2_ref, g2_ref, bb2_ref,
               w3_ref, b3_ref, g3_ref, bb3_ref,
               wo_ref, bo_ref, out_ref):
    xc = jnp.concatenate([zp_ref[...], xt_ref[...]], axis=1)
    h1 = jnp.dot(xc, w1_ref[...], precision=_PREC) + b1_ref[...]
    h1 = jax.nn.relu(_bn_rows(h1, g1_ref[...], bb1_ref[...]))
    h2 = jnp.dot(h1, w2_ref[...], precision=_PREC) + b2_ref[...]
    h2 = jax.nn.relu(_bn_rows(h2, g2_ref[...], bb2_ref[...]))
    h3 = jnp.dot(h2, w3_ref[...], precision=_PREC) + b3_ref[...]
    h3 = jax.nn.relu(_bn_rows(h3, g3_ref[...], bb3_ref[...]))
    out_ref[...] = jnp.dot(h3, wo_ref[...], precision=_PREC) + bo_ref[...]


def _head(zp, xt, p):
    return pl.pallas_call(
        _head_body,
        out_shape=jax.ShapeDtypeStruct((B, 1), jnp.float32),
    )(zp, xt, p['fc1_W'], p['fc1_b'], p['bnf1_g'], p['bnf1_b'],
      p['fc2_W'], p['fc2_b'], p['bnf2_g'], p['bnf2_b'],
      p['fc3_W'], p['fc3_b'], p['bnf3_g'], p['bnf3_b'],
      p['out_W'], p['out_b'])


def _bn_ncl(x, g, b):
    m = jnp.mean(x, (0, 2), keepdims=True)
    v = jnp.mean((x - m) ** 2, (0, 2), keepdims=True)
    return g[None, :, None] * (x - m) / jnp.sqrt(v + 1e-5) + b[None, :, None]


def _conv1d(x, W, b):
    y = jax.lax.conv_general_dilated(x, W, (1,), 'VALID',
                                     dimension_numbers=('NCH', 'OIH', 'NCH'))
    return y + b[None, :, None]


def _maxpool3(x):
    return jax.lax.reduce_window(x, -jnp.inf, jax.lax.max, (1, 1, 3), (1, 1, 3), 'VALID')


def kernel(x, edge_index, batch, target, params, eps):
    p = params
    src2 = jnp.concatenate([edge_index[0].astype(jnp.int32),
                            jnp.arange(N, dtype=jnp.int32)])
    dst2 = jnp.concatenate([edge_index[1].astype(jnp.int32),
                            jnp.arange(N, dtype=jnp.int32)])
    key = (dst2.astype(jnp.uint32) << jnp.uint32(16)) | src2.astype(jnp.uint32)
    key_s = jax.lax.sort(key)
    dst_s = (key_s >> jnp.uint32(16)).astype(jnp.int32)
    src_s = (key_s & jnp.uint32(0xFFFF)).astype(jnp.int32)

    row_start = jnp.searchsorted(dst_s, jnp.arange(N + 1, dtype=jnp.int32),
                                 method='sort')
    deg = jnp.diff(row_start).astype(jnp.float32)
    dinv = jax.lax.rsqrt(deg)

    npad = EPAD - (E + N)
    src_pad = jnp.concatenate([src_s, jnp.full((npad,), ZROW, jnp.int32)])
    dst_pad = jnp.concatenate([dst_s, jnp.full((npad,), jnp.int32(2**30), jnp.int32)])

    qb = jnp.searchsorted(dst_s, jnp.arange(0, NPAD + 1, QROWS, dtype=jnp.int32))
    qb = qb.at[NQ].set(EPAD)
    sblk = qb[:NQ] // KB
    nblk = (qb[1:] - sblk * KB + KB - 1) // KB
    tab = jnp.zeros((NQ, 16), jnp.int32)
    tab = tab.at[:, 0].set(sblk).at[:, 1].set(nblk).reshape(16 * NQ)

    dinv_c = dinv[:, None]

    def gcn_layer(h, W, b):
        u = dinv_c * jnp.dot(h, W, precision=_PREC)
        u = jnp.concatenate([u, jnp.zeros((UROWS - N, H), jnp.float32)])
        sfull = _propagate(u, src_pad, dst_pad, tab)
        return dinv_c * sfull[:N] + b

    identity = x @ p['res_W'] + p['res_b']
    h = jax.nn.relu(_bn_rows(gcn_layer(x, p['conv1_W'], p['conv1_b']), p['bn1_g'], p['bn1_b']))
    h = jax.nn.relu(_bn_rows(gcn_layer(h, p['conv2_W'], p['conv2_b']), p['bn2_g'], p['bn2_b']))
    h = jax.nn.relu(_bn_rows(gcn_layer(h, p['conv3_W'], p['conv3_b']), p['bn3_g'], p['bn3_b']))
    h = jax.nn.relu(_bn_rows(gcn_layer(h, p['conv4_W'], p['conv4_b']), p['bn4_g'], p['bn4_b']) + identity)
    mu = h @ p['mu_W'] + p['mu_b']
    logvar = h @ p['lv_W'] + p['lv_b']
    z = mu + eps * jnp.exp(0.5 * logvar)
    zp = jax.ops.segment_sum(z, batch, num_segments=B)
    m = jnp.mean(zp, -1, keepdims=True)
    v = jnp.mean((zp - m) ** 2, -1, keepdims=True)
    zp = p['ln_g'] * (zp - m) / jnp.sqrt(v + 1e-5) + p['ln_b']

    t = target[:, None, :]
    c = _maxpool3(jax.nn.relu(_bn_ncl(_conv1d(t, p['cxt1_W'], p['cxt1_b']), p['bnxt1_g'], p['bnxt1_b'])))
    c = _maxpool3(jax.nn.relu(_bn_ncl(_conv1d(c, p['cxt2_W'], p['cxt2_b']), p['bnxt2_g'], p['bnxt2_b'])))
    c = _maxpool3(jax.nn.relu(_bn_ncl(_conv1d(c, p['cxt3_W'], p['cxt3_b']), p['bnxt3_g'], p['bnxt3_b'])))
    xt = c.reshape(c.shape[0], -1) @ p['fc1xt_W'] + p['fc1xt_b']

    out = _head(zp, xt, p)
    return (out, zp)


# speedup vs baseline: 1.9079x; 1.9079x over previous
import functools

import jax
import jax.numpy as jnp
from jax import lax
from jax.experimental import pallas as pl
from jax.experimental.pallas import tpu as pltpu
from jax.experimental.pallas import tpu_sc as plsc

N = 50000
E = 800000
B = 256
L = 730
D_IN = 78
H = 128

_PREC = jax.lax.Precision.HIGHEST

QROWS = 8448
NQ = 6
NPAD = NQ * QROWS
UROWS = 50048
ZROW = 50000
TRASH = QROWS
BUFROWS = QROWS + 16
KB = 128
EPAD = 850048
NBLK = EPAD // KB
TROWS = QROWS // 16
ZROWS = 64
ZCHUNKS = [(o, min(ZROWS, TROWS - o)) for o in range(0, TROWS, ZROWS)]


def _propagate_body(u_hbm, srcs_hbm, dsts_hbm, tab_hbm, out_hbm,
                    tab_v, src_v, dst_v, idxl_v, rows_v, zero_v, buf_sh, sem):
    c = lax.axis_index("c")
    s = lax.axis_index("s")
    pltpu.sync_copy(tab_hbm, tab_v)
    tabs = [tab_v[pl.ds(16 * q, 16)] for q in range(NQ)]
    zvec = jnp.zeros((16,), jnp.float32)

    @pl.loop(0, ZROWS)
    def _(r):
        @pl.loop(0, H, step=16)
        def _(f):
            zero_v[r, pl.ds(f, 16)] = zvec

    for qi in range(NQ // 2):
        q = (NQ // 2) * c + qi
        qbase = q * QROWS
        tq = jnp.where(c == 0, tabs[qi], tabs[NQ // 2 + qi])
        sblk = tq[0]
        nblk = tq[1]

        for zo, zn in ZCHUNKS:
            zoff = pl.multiple_of(s * TROWS + zo, 8)
            pltpu.sync_copy(zero_v.at[pl.ds(0, zn)],
                            buf_sh.at[pl.ds(zoff, zn)])

        plsc.subcore_barrier()

        nmine = (nblk - s + 15) // 16

        @pl.loop(0, nmine)
        def _(i):
            blk = sblk + s + i * 16
            off = pl.multiple_of(blk * KB, KB)
            pltpu.sync_copy(srcs_hbm.at[pl.ds(off, KB)], src_v)
            pltpu.sync_copy(dsts_hbm.at[pl.ds(off, KB)], dst_v)
            for j in range(KB // 16):
                d = dst_v[pl.ds(j * 16, 16)]
                in_q = (d >= qbase) & (d < qbase + QROWS)
                loc = jnp.where(in_q, d - qbase, TRASH)
                idxl_v[pl.ds(j * 16, 16)] = loc
            pltpu.async_copy(u_hbm.at[src_v], rows_v, sem).wait()
            pltpu.sync_copy(rows_v, buf_sh.at[idxl_v], add=True)

        plsc.subcore_barrier()

        pltpu.sync_copy(buf_sh.at[pl.ds(pl.multiple_of(s * TROWS, 8), TROWS)],
                        out_hbm.at[pl.ds(pl.multiple_of(qbase + s * TROWS, 8), TROWS)])


def _propagate(u, srcs, dsts, tab):
    mesh = plsc.VectorSubcoreMesh(core_axis_name="c", subcore_axis_name="s")
    kern = pl.kernel(
        _propagate_body,
        out_type=jax.ShapeDtypeStruct((NPAD, H), jnp.float32),
        mesh=mesh,
        scratch_types=[
            pltpu.VMEM((16 * NQ,), jnp.int32),
            pltpu.VMEM((KB,), jnp.int32),
            pltpu.VMEM((KB,), jnp.int32),
            pltpu.VMEM((KB,), jnp.int32),
            pltpu.VMEM((KB, H), jnp.float32),
            pltpu.VMEM((ZROWS, H), jnp.float32),
            pltpu.VMEM_SHARED((BUFROWS, H), jnp.float32),
            pltpu.SemaphoreType.DMA,
        ],
    )
    return kern(u, srcs, dsts, tab)


def _bn_rows(x, g, b):
    m = jnp.mean(x, 0)
    v = jnp.mean((x - m) ** 2, 0)
    return g * (x - m) / jnp.sqrt(v + 1e-5) + b


def _head_body(zp_ref, xt_ref, w1_ref, b1_ref, g1_ref, bb1_ref,
               w2_ref, b2_ref, g2_ref, bb2_ref,
               w3_ref, b3_ref, g3_ref, bb3_ref,
               wo_ref, bo_ref, out_ref):
    xc = jnp.concatenate([zp_ref[...], xt_ref[...]], axis=1)
    h1 = jnp.dot(xc, w1_ref[...], precision=_PREC) + b1_ref[...]
    h1 = jax.nn.relu(_bn_rows(h1, g1_ref[...], bb1_ref[...]))
    h2 = jnp.dot(h1, w2_ref[...], precision=_PREC) + b2_ref[...]
    h2 = jax.nn.relu(_bn_rows(h2, g2_ref[...], bb2_ref[...]))
    h3 = jnp.dot(h2, w3_ref[...], precision=_PREC) + b3_ref[...]
    h3 = jax.nn.relu(_bn_rows(h3, g3_ref[...], bb3_ref[...]))
    out_ref[...] = jnp.dot(h3, wo_ref[...], precision=_PREC) + bo_ref[...]


def _head(zp, xt, p):
    return pl.pallas_call(
        _head_body,
        out_shape=jax.ShapeDtypeStruct((B, 1), jnp.float32),
    )(zp, xt, p['fc1_W'], p['fc1_b'], p['bnf1_g'], p['bnf1_b'],
      p['fc2_W'], p['fc2_b'], p['bnf2_g'], p['bnf2_b'],
      p['fc3_W'], p['fc3_b'], p['bnf3_g'], p['bnf3_b'],
      p['out_W'], p['out_b'])


def _bn_ncl(x, g, b):
    m = jnp.mean(x, (0, 2), keepdims=True)
    v = jnp.mean((x - m) ** 2, (0, 2), keepdims=True)
    return g[None, :, None] * (x - m) / jnp.sqrt(v + 1e-5) + b[None, :, None]


def _conv1d(x, W, b):
    y = jax.lax.conv_general_dilated(x, W, (1,), 'VALID',
                                     dimension_numbers=('NCH', 'OIH', 'NCH'))
    return y + b[None, :, None]


def _maxpool3(x):
    return jax.lax.reduce_window(x, -jnp.inf, jax.lax.max, (1, 1, 3), (1, 1, 3), 'VALID')


def kernel(x, edge_index, batch, target, params, eps):
    p = params
    src2 = jnp.concatenate([edge_index[0].astype(jnp.int32),
                            jnp.arange(N, dtype=jnp.int32)])
    dst2 = jnp.concatenate([edge_index[1].astype(jnp.int32),
                            jnp.arange(N, dtype=jnp.int32)])
    key = (dst2.astype(jnp.uint32) << jnp.uint32(16)) | src2.astype(jnp.uint32)
    key_s = jax.lax.sort(key)
    dst_s = (key_s >> jnp.uint32(16)).astype(jnp.int32)
    src_s = (key_s & jnp.uint32(0xFFFF)).astype(jnp.int32)


    npad = EPAD - (E + N)
    src_pad = jnp.concatenate([src_s, jnp.full((npad,), ZROW, jnp.int32)])
    dst_pad = jnp.concatenate([dst_s, jnp.full((npad,), jnp.int32(2**30), jnp.int32)])

    qb = jnp.searchsorted(dst_s, jnp.arange(0, NPAD + 1, QROWS, dtype=jnp.int32))
    qb = qb.at[NQ].set(EPAD)
    sblk = qb[:NQ] // KB
    nblk = (qb[1:] - sblk * KB + KB - 1) // KB
    tab = jnp.zeros((NQ, 16), jnp.int32)
    tab = tab.at[:, 0].set(sblk).at[:, 1].set(nblk).reshape(16 * NQ)

    ones_u = jnp.ones((UROWS, H), jnp.float32)
    deg = _propagate(ones_u, src_pad, dst_pad, tab)[:N, :1]
    dinv_c = jax.lax.rsqrt(deg)

    def gcn_layer(h, W, b):
        u = dinv_c * jnp.dot(h, W, precision=_PREC)
        u = jnp.concatenate([u, jnp.zeros((UROWS - N, H), jnp.float32)])
        sfull = _propagate(u, src_pad, dst_pad, tab)
        return dinv_c * sfull[:N] + b

    identity = x @ p['res_W'] + p['res_b']
    h = jax.nn.relu(_bn_rows(gcn_layer(x, p['conv1_W'], p['conv1_b']), p['bn1_g'], p['bn1_b']))
    h = jax.nn.relu(_bn_rows(gcn_layer(h, p['conv2_W'], p['conv2_b']), p['bn2_g'], p['bn2_b']))
    h = jax.nn.relu(_bn_rows(gcn_layer(h, p['conv3_W'], p['conv3_b']), p['bn3_g'], p['bn3_b']))
    h = jax.nn.relu(_bn_rows(gcn_layer(h, p['conv4_W'], p['conv4_b']), p['bn4_g'], p['bn4_b']) + identity)
    mu = h @ p['mu_W'] + p['mu_b']
    logvar = h @ p['lv_W'] + p['lv_b']
    z = mu + eps * jnp.exp(0.5 * logvar)
    zp = jax.ops.segment_sum(z, batch, num_segments=B)
    m = jnp.mean(zp, -1, keepdims=True)
    v = jnp.mean((zp - m) ** 2, -1, keepdims=True)
    zp = p['ln_g'] * (zp - m) / jnp.sqrt(v + 1e-5) + p['ln_b']

    t = target[:, None, :]
    c = _maxpool3(jax.nn.relu(_bn_ncl(_conv1d(t, p['cxt1_W'], p['cxt1_b']), p['bnxt1_g'], p['bnxt1_b'])))
    c = _maxpool3(jax.nn.relu(_bn_ncl(_conv1d(c, p['cxt2_W'], p['cxt2_b']), p['bnxt2_g'], p['bnxt2_b'])))
    c = _maxpool3(jax.nn.relu(_bn_ncl(_conv1d(c, p['cxt3_W'], p['cxt3_b']), p['bnxt3_g'], p['bnxt3_b'])))
    xt = c.reshape(c.shape[0], -1) @ p['fc1xt_W'] + p['fc1xt_b']

    out = _head(zp, xt, p)
    return (out, zp)


# speedup vs baseline: 1.9910x; 1.0435x over previous
import functools

import jax
import jax.numpy as jnp
from jax import lax
from jax.experimental import pallas as pl
from jax.experimental.pallas import tpu as pltpu
from jax.experimental.pallas import tpu_sc as plsc

N = 50000
E = 800000
B = 256
L = 730
D_IN = 78
H = 128

_PREC = jax.lax.Precision.HIGHEST

QROWS = 8448
NQ = 6
NPAD = NQ * QROWS
UROWS = 50048
ZROW = 50000
TRASH = QROWS
BUFROWS = QROWS + 16
KB = 128
EPAD = 850048
NBLK = EPAD // KB
TROWS = QROWS // 16
ZROWS = 64
ZCHUNKS = [(o, min(ZROWS, TROWS - o)) for o in range(0, TROWS, ZROWS)]


def _propagate_body(u_hbm, srcs_hbm, dsts_hbm, tab_hbm, out_hbm,
                    tab_v, src_v, dst_v, idxl_v, rows_v, zero_v, buf_sh, sem):
    c = lax.axis_index("c")
    s = lax.axis_index("s")
    pltpu.sync_copy(tab_hbm, tab_v)
    tabs = [tab_v[pl.ds(16 * q, 16)] for q in range(NQ)]
    zvec = jnp.zeros((16,), jnp.float32)

    @pl.loop(0, ZROWS)
    def _(r):
        @pl.loop(0, H, step=16)
        def _(f):
            zero_v[r, pl.ds(f, 16)] = zvec

    for qi in range(NQ // 2):
        q = (NQ // 2) * c + qi
        qbase = q * QROWS
        tq = jnp.where(c == 0, tabs[qi], tabs[NQ // 2 + qi])
        sblk = tq[0]
        nblk = tq[1]

        for zo, zn in ZCHUNKS:
            zoff = pl.multiple_of(s * TROWS + zo, 8)
            pltpu.sync_copy(zero_v.at[pl.ds(0, zn)],
                            buf_sh.at[pl.ds(zoff, zn)])

        plsc.subcore_barrier()

        nmine = (nblk - s + 15) // 16

        @pl.loop(0, nmine)
        def _(i):
            blk = sblk + s + i * 16
            off = pl.multiple_of(blk * KB, KB)
            pltpu.sync_copy(srcs_hbm.at[pl.ds(off, KB)], src_v)
            pltpu.sync_copy(dsts_hbm.at[pl.ds(off, KB)], dst_v)
            for j in range(KB // 16):
                d = dst_v[pl.ds(j * 16, 16)]
                in_q = (d >= qbase) & (d < qbase + QROWS)
                loc = jnp.where(in_q, d - qbase, TRASH)
                idxl_v[pl.ds(j * 16, 16)] = loc
            pltpu.async_copy(u_hbm.at[src_v], rows_v, sem).wait()
            pltpu.sync_copy(rows_v, buf_sh.at[idxl_v], add=True)

        plsc.subcore_barrier()

        pltpu.sync_copy(buf_sh.at[pl.ds(pl.multiple_of(s * TROWS, 8), TROWS)],
                        out_hbm.at[pl.ds(pl.multiple_of(qbase + s * TROWS, 8), TROWS)])


def _propagate(u, srcs, dsts, tab):
    mesh = plsc.VectorSubcoreMesh(core_axis_name="c", subcore_axis_name="s")
    kern = pl.kernel(
        _propagate_body,
        out_type=jax.ShapeDtypeStruct((NPAD, H), jnp.float32),
        mesh=mesh,
        scratch_types=[
            pltpu.VMEM((16 * NQ,), jnp.int32),
            pltpu.VMEM((KB,), jnp.int32),
            pltpu.VMEM((KB,), jnp.int32),
            pltpu.VMEM((KB,), jnp.int32),
            pltpu.VMEM((KB, H), jnp.float32),
            pltpu.VMEM((ZROWS, H), jnp.float32),
            pltpu.VMEM_SHARED((BUFROWS, H), jnp.float32),
            pltpu.SemaphoreType.DMA,
        ],
    )
    return kern(u, srcs, dsts, tab)


RB = 1024
NRB = (UROWS + RB - 1) // RB
NBP = NRB * RB


def _row_mask(pid, val, fill=0.0):
    rows = jax.lax.broadcasted_iota(jnp.int32, val.shape, 0) + pid * RB
    return jnp.where(rows < N, val, fill)


def _mm_scale_body(dinv_ref, h_ref, w_ref, o_ref):
    pid = pl.program_id(0)
    val = dinv_ref[...] * jnp.dot(h_ref[...], w_ref[...], precision=_PREC)
    o_ref[...] = _row_mask(pid, val)


def _mm_scale(dinvp, h, w):
    d = h.shape[1]
    return pl.pallas_call(
        _mm_scale_body,
        grid=(NRB,),
        in_specs=[pl.BlockSpec((RB, 1), lambda i: (i, 0)),
                  pl.BlockSpec((RB, d), lambda i: (i, 0)),
                  pl.BlockSpec((d, H), lambda i: (0, 0))],
        out_specs=pl.BlockSpec((RB, H), lambda i: (i, 0)),
        out_shape=jax.ShapeDtypeStruct((UROWS, H), jnp.float32),
    )(dinvp, h, w)


def _stats_body(dinv_ref, s_ref, stat_ref):
    pid = pl.program_id(0)
    a = _row_mask(pid, dinv_ref[...] * s_ref[...])
    ps = jnp.stack([jnp.sum(a, 0), jnp.sum(a * a, 0)])
    prev = jnp.where(pid == 0, jnp.zeros((2, H), jnp.float32), stat_ref[...])
    stat_ref[...] = prev + ps


def _stats(dinvp, s):
    return pl.pallas_call(
        _stats_body,
        grid=(NRB,),
        in_specs=[pl.BlockSpec((RB, 1), lambda i: (i, 0)),
                  pl.BlockSpec((RB, H), lambda i: (i, 0))],
        out_specs=pl.BlockSpec((2, H), lambda i: (0, 0)),
        out_shape=jax.ShapeDtypeStruct((2, H), jnp.float32),
    )(dinvp, s)


def _bn_from_stats(a, stat_ref, g_ref, bb_ref):
    m = stat_ref[0:1, :] / N
    v = stat_ref[1:2, :] / N - m * m
    return g_ref[...] * (a - m) * jax.lax.rsqrt(v + 1e-5) + bb_ref[...]


def _apply_mm_body(stat_ref, dinv_ref, s_ref, g_ref, bb_ref, w_ref, o_ref):
    pid = pl.program_id(0)
    a = dinv_ref[...] * s_ref[...]
    h = jax.nn.relu(_bn_from_stats(a, stat_ref, g_ref, bb_ref))
    val = dinv_ref[...] * jnp.dot(h, w_ref[...], precision=_PREC)
    o_ref[...] = _row_mask(pid, val)


def _apply_mm(stat, dinvp, s, g, bb, w):
    return pl.pallas_call(
        _apply_mm_body,
        grid=(NRB,),
        in_specs=[pl.BlockSpec((2, H), lambda i: (0, 0)),
                  pl.BlockSpec((RB, 1), lambda i: (i, 0)),
                  pl.BlockSpec((RB, H), lambda i: (i, 0)),
                  pl.BlockSpec((1, H), lambda i: (0, 0)),
                  pl.BlockSpec((1, H), lambda i: (0, 0)),
                  pl.BlockSpec((H, H), lambda i: (0, 0))],
        out_specs=pl.BlockSpec((RB, H), lambda i: (i, 0)),
        out_shape=jax.ShapeDtypeStruct((UROWS, H), jnp.float32),
    )(stat, dinvp, s, g, bb, w)


def _final_z_body(stat_ref, dinv_ref, s_ref, g_ref, bb_ref, x_ref, rw_ref,
                  rb_ref, mw_ref, mb_ref, lw_ref, lb_ref, eps_ref, z_ref):
    pid = pl.program_id(0)
    a = dinv_ref[...] * s_ref[...]
    ident = jnp.dot(x_ref[...], rw_ref[...], precision=_PREC) + rb_ref[...]
    h = jax.nn.relu(_bn_from_stats(a, stat_ref, g_ref, bb_ref) + ident)
    mu = jnp.dot(h, mw_ref[...], precision=_PREC) + mb_ref[...]
    lv = jnp.dot(h, lw_ref[...], precision=_PREC) + lb_ref[...]
    val = mu + eps_ref[...] * jnp.exp(0.5 * lv)
    z_ref[...] = _row_mask(pid, val)


def _final_z(stat, dinvp, s, g, bb, x, rw, rb, mw, mb, lw, lb, eps):
    return pl.pallas_call(
        _final_z_body,
        grid=(NRB,),
        in_specs=[pl.BlockSpec((2, H), lambda i: (0, 0)),
                  pl.BlockSpec((RB, 1), lambda i: (i, 0)),
                  pl.BlockSpec((RB, H), lambda i: (i, 0)),
                  pl.BlockSpec((1, H), lambda i: (0, 0)),
                  pl.BlockSpec((1, H), lambda i: (0, 0)),
                  pl.BlockSpec((RB, D_IN), lambda i: (i, 0)),
                  pl.BlockSpec((D_IN, H), lambda i: (0, 0)),
                  pl.BlockSpec((1, H), lambda i: (0, 0)),
                  pl.BlockSpec((H, H), lambda i: (0, 0)),
                  pl.BlockSpec((1, H), lambda i: (0, 0)),
                  pl.BlockSpec((H, H), lambda i: (0, 0)),
                  pl.BlockSpec((1, H), lambda i: (0, 0)),
                  pl.BlockSpec((RB, H), lambda i: (i, 0))],
        out_specs=pl.BlockSpec((RB, H), lambda i: (i, 0)),
        out_shape=jax.ShapeDtypeStruct((UROWS, H), jnp.float32),
    )(stat, dinvp, s, g, bb, x, rw, rb, mw, mb, lw, lb, eps)


def _pool_ln_body(batch_ref, z_ref, g_ref, bb_ref, zp_ref, acc_ref):
    pid = pl.program_id(0)
    bvec = batch_ref[0, 0, :]
    gid = jax.lax.broadcasted_iota(jnp.int32, (RB, B), 1)
    onehot = (bvec[:, None] == gid).astype(jnp.float32)
    zblk = _row_mask(pid, z_ref[...])
    contrib = jax.lax.dot_general(onehot, zblk, (((0,), (0,)), ((), ())),
                                  precision=_PREC)
    prev = jnp.where(pid == 0, jnp.zeros((B, H), jnp.float32), acc_ref[...])
    acc_ref[...] = prev + contrib

    @pl.when(pid == NRB - 1)
    def _():
        zp = acc_ref[...]
        m = jnp.mean(zp, -1, keepdims=True)
        v = jnp.mean((zp - m) ** 2, -1, keepdims=True)
        zp_ref[...] = g_ref[...] * (zp - m) / jnp.sqrt(v + 1e-5) + bb_ref[...]


def _pool_ln(batchp3, z, g, bb):
    return pl.pallas_call(
        _pool_ln_body,
        grid=(NRB,),
        in_specs=[pl.BlockSpec((1, 1, RB), lambda i: (i, 0, 0)),
                  pl.BlockSpec((RB, H), lambda i: (i, 0)),
                  pl.BlockSpec((1, H), lambda i: (0, 0)),
                  pl.BlockSpec((1, H), lambda i: (0, 0))],
        out_specs=pl.BlockSpec((B, H), lambda i: (0, 0)),
        out_shape=jax.ShapeDtypeStruct((B, H), jnp.float32),
        scratch_shapes=[pltpu.VMEM((B, H), jnp.float32)],
    )(batchp3, z, g, bb)


def _bn_rows(x, g, b):
    m = jnp.mean(x, 0)
    v = jnp.mean((x - m) ** 2, 0)
    return g * (x - m) / jnp.sqrt(v + 1e-5) + b


def _head_body(zp_ref, xt_ref, w1_ref, b1_ref, g1_ref, bb1_ref,
               w2_ref, b2_ref, g2_ref, bb2_ref,
               w3_ref, b3_ref, g3_ref, bb3_ref,
               wo_ref, bo_ref, out_ref):
    xc = jnp.concatenate([zp_ref[...], xt_ref[...]], axis=1)
    h1 = jnp.dot(xc, w1_ref[...], precision=_PREC) + b1_ref[...]
    h1 = jax.nn.relu(_bn_rows(h1, g1_ref[...], bb1_ref[...]))
    h2 = jnp.dot(h1, w2_ref[...], precision=_PREC) + b2_ref[...]
    h2 = jax.nn.relu(_bn_rows(h2, g2_ref[...], bb2_ref[...]))
    h3 = jnp.dot(h2, w3_ref[...], precision=_PREC) + b3_ref[...]
    h3 = jax.nn.relu(_bn_rows(h3, g3_ref[...], bb3_ref[...]))
    out_ref[...] = jnp.dot(h3, wo_ref[...], precision=_PREC) + bo_ref[...]


def _head(zp, xt, p):
    return pl.pallas_call(
        _head_body,
        out_shape=jax.ShapeDtypeStruct((B, 1), jnp.float32),
    )(zp, xt, p['fc1_W'], p['fc1_b'], p['bnf1_g'], p['bnf1_b'],
      p['fc2_W'], p['fc2_b'], p['bnf2_g'], p['bnf2_b'],
      p['fc3_W'], p['fc3_b'], p['bnf3_g'], p['bnf3_b'],
      p['out_W'], p['out_b'])


def _bn_ncl(x, g, b):
    m = jnp.mean(x, (0, 2), keepdims=True)
    v = jnp.mean((x - m) ** 2, (0, 2), keepdims=True)
    return g[None, :, None] * (x - m) / jnp.sqrt(v + 1e-5) + b[None, :, None]


def _conv1d(x, W, b):
    y = jax.lax.conv_general_dilated(x, W, (1,), 'VALID',
                                     dimension_numbers=('NCH', 'OIH', 'NCH'))
    return y + b[None, :, None]


def _maxpool3(x):
    return jax.lax.reduce_window(x, -jnp.inf, jax.lax.max, (1, 1, 3), (1, 1, 3), 'VALID')


def kernel(x, edge_index, batch, target, params, eps):
    p = params
    src2 = jnp.concatenate([edge_index[0].astype(jnp.int32),
                            jnp.arange(N, dtype=jnp.int32)])
    dst2 = jnp.concatenate([edge_index[1].astype(jnp.int32),
                            jnp.arange(N, dtype=jnp.int32)])
    key = (dst2.astype(jnp.uint32) << jnp.uint32(16)) | src2.astype(jnp.uint32)
    key_s = jax.lax.sort(key)
    dst_s = (key_s >> jnp.uint32(16)).astype(jnp.int32)
    src_s = (key_s & jnp.uint32(0xFFFF)).astype(jnp.int32)


    npad = EPAD - (E + N)
    src_pad = jnp.concatenate([src_s, jnp.full((npad,), ZROW, jnp.int32)])
    dst_pad = jnp.concatenate([dst_s, jnp.full((npad,), jnp.int32(2**30), jnp.int32)])

    qb = jnp.searchsorted(dst_s, jnp.arange(0, NPAD + 1, QROWS, dtype=jnp.int32))
    qb = qb.at[NQ].set(EPAD)
    sblk = qb[:NQ] // KB
    nblk = (qb[1:] - sblk * KB + KB - 1) // KB
    tab = jnp.zeros((NQ, 16), jnp.int32)
    tab = tab.at[:, 0].set(sblk).at[:, 1].set(nblk).reshape(16 * NQ)

    ones_u = jnp.ones((UROWS, H), jnp.float32)
    deg = _propagate(ones_u, src_pad, dst_pad, tab)[:N, :1]
    dinvp = jnp.concatenate([jax.lax.rsqrt(deg),
                             jnp.zeros((UROWS - N, 1), jnp.float32)])

    def r1(a):
        return a.reshape(1, H)

    u = _mm_scale(dinvp, x, p['conv1_W'])
    s1 = _propagate(u, src_pad, dst_pad, tab)[:UROWS]
    st1 = _stats(dinvp, s1)
    u = _apply_mm(st1, dinvp, s1, r1(p['bn1_g']), r1(p['bn1_b']), p['conv2_W'])
    s2 = _propagate(u, src_pad, dst_pad, tab)[:UROWS]
    st2 = _stats(dinvp, s2)
    u = _apply_mm(st2, dinvp, s2, r1(p['bn2_g']), r1(p['bn2_b']), p['conv3_W'])
    s3 = _propagate(u, src_pad, dst_pad, tab)[:UROWS]
    st3 = _stats(dinvp, s3)
    u = _apply_mm(st3, dinvp, s3, r1(p['bn3_g']), r1(p['bn3_b']), p['conv4_W'])
    s4 = _propagate(u, src_pad, dst_pad, tab)[:UROWS]
    st4 = _stats(dinvp, s4)
    z = _final_z(st4, dinvp, s4, r1(p['bn4_g']), r1(p['bn4_b']), x,
                 p['res_W'], r1(p['res_b']), p['mu_W'], r1(p['mu_b']),
                 p['lv_W'], r1(p['lv_b']), eps)

    batchp3 = jnp.concatenate([batch.astype(jnp.int32),
                               jnp.full((NBP - N,), B, jnp.int32)]).reshape(NRB, 1, RB)
    zp = _pool_ln(batchp3, z, r1(p['ln_g']), r1(p['ln_b']))

    t = target[:, None, :]
    c = _maxpool3(jax.nn.relu(_bn_ncl(_conv1d(t, p['cxt1_W'], p['cxt1_b']), p['bnxt1_g'], p['bnxt1_b'])))
    c = _maxpool3(jax.nn.relu(_bn_ncl(_conv1d(c, p['cxt2_W'], p['cxt2_b']), p['bnxt2_g'], p['bnxt2_b'])))
    c = _maxpool3(jax.nn.relu(_bn_ncl(_conv1d(c, p['cxt3_W'], p['cxt3_b']), p['bnxt3_g'], p['bnxt3_b'])))
    xt = c.reshape(c.shape[0], -1) @ p['fc1xt_W'] + p['fc1xt_b']

    out = _head(zp, xt, p)
    return (out, zp)
